# baseline (speedup 1.0000x reference)
import functools

import jax
import jax.numpy as jnp
from jax import lax
from jax.experimental import pallas as pl
from jax.experimental.pallas import tpu as pltpu
from jax.experimental.pallas import tpu_sc as plsc

NC = 2
NS = 16
NW = NC * NS

SLABS = 32


def _sc_gather(table, idx2d, *, b0, b1, d):
    spw = b0 // NW
    nch = spw // SLABS
    mesh = plsc.VectorSubcoreMesh(
        core_axis_name="c", subcore_axis_name="s", num_cores=NC, num_subcores=NS
    )

    @functools.partial(
        pl.kernel,
        out_type=jax.ShapeDtypeStruct((b0, b1, d), jnp.float32),
        mesh=mesh,
        compiler_params=pltpu.CompilerParams(use_tc_tiling_on_sc=False),
        scratch_types=[
            pltpu.VMEM((spw, b1), jnp.int32),
            pltpu.VMEM((SLABS, b1, d), jnp.float32),
            pltpu.VMEM((SLABS, b1, d), jnp.float32),
            pltpu.SemaphoreType.DMA,
            pltpu.SemaphoreType.DMA,
            pltpu.SemaphoreType.DMA,
            pltpu.SemaphoreType.DMA,
        ],
    )
    def grab(
        table_hbm, idx_hbm, out_hbm, idx_v, buf0, buf1, gs0, gs1, ss0, ss1
    ):
        wid = lax.axis_index("s") * NC + lax.axis_index("c")
        base = wid * spw
        pltpu.sync_copy(idx_hbm.at[pl.ds(base, spw)], idx_v)

        bufs = (buf0, buf1)
        gsems = (gs0, gs1)
        ssems = (ss0, ss1)

        def store_copy(j, b):
            return pltpu.make_async_copy(
                bufs[b], out_hbm.at[pl.ds(base + j * SLABS, SLABS)], ssems[b]
            )

        def gather_start(j, b):
            for g in range(SLABS):
                pltpu.make_async_copy(
                    table_hbm.at[idx_v.at[j * SLABS + g]], bufs[b].at[g], gsems[b]
                ).start()

        def gather_wait(b):
            for g in range(SLABS):
                pltpu.make_async_copy(
                    table_hbm.at[idx_v.at[0]], bufs[b].at[g], gsems[b]
                ).wait()

        gather_start(0, 0)

        def outer(g, carry):
            for b in range(2):
                j = g * 2 + b

                @pl.when(j + 1 < nch)
                def _():
                    @pl.when(j >= 1)
                    def _():
                        store_copy(0, 1 - b).wait()

                    gather_start(j + 1, 1 - b)

                gather_wait(b)
                store_copy(j, b).start()
            return carry

        lax.fori_loop(0, nch // 2, outer, 0, unroll=False)
        store_copy(0, 0).wait()
        store_copy(0, 1).wait()

    return grab(table, idx2d)


def kernel(arg0_1, arg1_1):
    b0, b1 = arg0_1.shape
    v, d = arg1_1.shape
    idx2d = arg0_1.astype(jnp.int32)
    return (_sc_gather(arg1_1, idx2d, b0=b0, b1=b1, d=d),)
